# baseline (speedup 1.0000x reference)
import jax
import jax.numpy as jnp
from jax.experimental import pallas as pl
from jax.experimental.pallas import tpu as pltpu
from jax.experimental.pallas import tpu_sc as plsc

B, S, D, DFF, E, K = 1, 2048, 768, 2048, 8, 2
N = B * S
NK = N * K
DH = D // 2
TILE = 256
PADDED = 6144
GRID_TILES = PADDED // TILE


def _sc_mesh():
    return plsc.VectorSubcoreMesh(core_axis_name="c", subcore_axis_name="s")


def _pack_bf16(a):
    ab = a.astype(jnp.bfloat16)
    hi = jax.lax.bitcast_convert_type(ab[:, :DH], jnp.uint16)
    lo = jax.lax.bitcast_convert_type(ab[:, DH:], jnp.uint16)
    word = (hi.astype(jnp.uint32) << 16) | lo.astype(jnp.uint32)
    return jax.lax.bitcast_convert_type(word, jnp.int32)


def _unpack_bf16(w):
    u = jax.lax.bitcast_convert_type(w, jnp.uint32)
    hi = jax.lax.bitcast_convert_type(
        (u >> 16).astype(jnp.uint16), jnp.bfloat16)
    lo = jax.lax.bitcast_convert_type(
        (u & 0xFFFF).astype(jnp.uint16), jnp.bfloat16)
    return jnp.concatenate([hi, lo], axis=1)


def _router_body(x_ref, wr_ref, pk_ref, g_ref, meta_ref, x32_ref):
    xf = x_ref[...]
    x32_ref[...] = _pack_bf16(xf)
    logits = jax.lax.dot_general(
        xf, wr_ref[...], (((1,), (1,)), ((), ())),
        preferred_element_type=jnp.float32,
        precision=jax.lax.Precision.DEFAULT)
    m = jnp.max(logits, axis=1, keepdims=True)
    p = jnp.exp(logits - m)
    probs = p / jnp.sum(p, axis=1, keepdims=True)

    lane = jax.lax.broadcasted_iota(jnp.int32, (N, E), 1)
    m1 = jnp.max(probs, axis=1, keepdims=True)
    i1 = jnp.min(jnp.where(probs == m1, lane, E), axis=1, keepdims=True)
    probs2 = jnp.where(lane == i1, -1.0, probs)
    m2 = jnp.max(probs2, axis=1, keepdims=True)
    i2 = jnp.min(jnp.where(probs2 == m2, lane, E), axis=1, keepdims=True)
    denom = m1 + m2 + 1e-9
    g_ref[...] = jnp.concatenate([m1 / denom, m2 / denom], axis=1)

    oh1 = (lane == i1).astype(jnp.float32)
    oh2 = (lane == i2).astype(jnp.float32)
    a_all = oh1 + oh2

    r_i = jax.lax.broadcasted_iota(jnp.int32, (N, N), 0)
    c_i = jax.lax.broadcasted_iota(jnp.int32, (N, N), 1)
    ltri = (r_i > c_i).astype(jnp.bfloat16)
    cum = jax.lax.dot_general(
        ltri, a_all.astype(jnp.bfloat16), (((1,), (0,)), ((), ())),
        preferred_element_type=jnp.float32)

    counts = jnp.sum(a_all, axis=0, keepdims=True)
    pc = jnp.ceil(counts * (1.0 / TILE)) * TILE
    e_r = jax.lax.broadcasted_iota(jnp.int32, (E, E), 0)
    e_c = jax.lax.broadcasted_iota(jnp.int32, (E, E), 1)
    up = (e_r < e_c).astype(jnp.float32)
    starts = jax.lax.dot_general(
        pc, up, (((1,), (0,)), ((), ())),
        preferred_element_type=jnp.float32,
        precision=jax.lax.Precision.HIGHEST)

    rank1 = jnp.sum(cum * oh1, axis=1, keepdims=True)
    rank2 = jnp.sum(cum * oh2, axis=1, keepdims=True)
    s1 = jnp.sum(starts * oh1, axis=1, keepdims=True)
    s2 = jnp.sum(starts * oh2, axis=1, keepdims=True)
    pk = jnp.concatenate([s1 + rank1, s2 + rank2], axis=1)
    pk_ref[...] = pk.astype(jnp.int32)

    ones_col = jnp.ones((N, 1), jnp.float32)
    counts_col = jax.lax.dot_general(
        a_all, ones_col, (((0,), (0,)), ((), ())),
        preferred_element_type=jnp.float32)
    pc_col = jnp.ceil(counts_col * (1.0 / TILE)) * TILE
    low = (e_r > e_c).astype(jnp.float32)
    starts_col = jax.lax.dot_general(
        low, pc_col, (((1,), (0,)), ((), ())),
        preferred_element_type=jnp.float32,
        precision=jax.lax.Precision.HIGHEST)
    total = jnp.sum(pc_col)

    tl = jax.lax.broadcasted_iota(jnp.int32, (E, 128), 1).astype(
        jnp.float32) * TILE
    s_cl = jnp.minimum(tl, total - TILE)
    te = jnp.sum((s_cl >= starts_col).astype(jnp.float32), axis=0,
                 keepdims=True) - 1.0
    active = (jax.lax.broadcasted_iota(jnp.int32, (1, 128), 1).astype(
        jnp.float32) * TILE < total).astype(jnp.int32)
    meta_ref[...] = jnp.concatenate([te.astype(jnp.int32), active], axis=0)


def _route(x_flat, Wr):
    return pl.pallas_call(
        _router_body,
        out_shape=(jax.ShapeDtypeStruct((N, K), jnp.int32),
                   jax.ShapeDtypeStruct((N, K), jnp.float32),
                   jax.ShapeDtypeStruct((2, 128), jnp.int32),
                   jax.ShapeDtypeStruct((N, DH), jnp.int32)),
        in_specs=[pl.BlockSpec((N, D), lambda: (0, 0)),
                  pl.BlockSpec((E, D), lambda: (0, 0))],
        out_specs=(pl.BlockSpec((N, K), lambda: (0, 0)),
                   pl.BlockSpec((N, K), lambda: (0, 0)),
                   pl.BlockSpec((2, 128), lambda: (0, 0)),
                   pl.BlockSpec((N, DH), lambda: (0, 0))),
    )(x_flat, Wr)


_DISP_W = 128


def _sc_dispatch(x32, t_row, p_row):
    @pl.kernel(
        out_type=jax.ShapeDtypeStruct((PADDED, DH), jnp.int32),
        mesh=_sc_mesh(),
        scratch_types=[pltpu.VMEM((_DISP_W, DH), jnp.int32)])
    def k(x_hbm, t_hbm, p_hbm, xs_hbm, buf):
        def body(t_vmem, p_vmem):
            pltpu.sync_copy(x_hbm.at[t_vmem.at[0]], buf)
            pltpu.sync_copy(buf, xs_hbm.at[p_vmem.at[0]])

        pltpu.emit_pipeline(
            body,
            grid=(NK // _DISP_W,),
            in_specs=[pl.BlockSpec((1, _DISP_W), lambda i: (0, i)),
                      pl.BlockSpec((1, _DISP_W), lambda i: (0, i))],
            out_specs=[],
            core_axis_name=("c", "s"),
            dimension_semantics=(pltpu.PARALLEL,),
        )(t_hbm, p_hbm)

    return k(x32, t_row, p_row)


_FCH = 2
_FSZ = DFF // _FCH


def _ffn_body(meta_ref, xs_ref, w1_ref, b1_ref, w2_ref, b2_ref, out_ref,
              acc_ref):
    i = pl.program_id(0)
    j = pl.program_id(1)

    @pl.when(meta_ref[1, i] == 1)
    def _():
        xb = _unpack_bf16(xs_ref[...])
        w1 = w1_ref[0].astype(jnp.bfloat16)
        h = jax.lax.dot_general(
            xb, w1, (((1,), (0,)), ((), ())),
            preferred_element_type=jnp.float32)
        h = h + b1_ref[0]
        h = 0.5 * h * (1.0 + jax.lax.erf(h * 0.7071067811865476))
        w2 = w2_ref[0].astype(jnp.bfloat16)
        y = jax.lax.dot_general(
            h.astype(jnp.bfloat16), w2, (((1,), (0,)), ((), ())),
            preferred_element_type=jnp.float32)

        @pl.when(j == 0)
        def _():
            acc_ref[...] = y + b2_ref[0]

        @pl.when(j != 0)
        def _():
            acc = acc_ref[...] + y
            acc_ref[...] = acc

            @pl.when(j == _FCH - 1)
            def _():
                out_ref[...] = _pack_bf16(acc)


def _ffn(meta, xs32, W1, b1, W2, b2):
    grid_spec = pltpu.PrefetchScalarGridSpec(
        num_scalar_prefetch=1,
        grid=(GRID_TILES, _FCH),
        in_specs=[
            pl.BlockSpec((TILE, DH), lambda i, j, m: (i, 0)),
            pl.BlockSpec((1, D, _FSZ), lambda i, j, m: (m[0, i], 0, j)),
            pl.BlockSpec((1, 1, _FSZ), lambda i, j, m: (m[0, i], 0, j)),
            pl.BlockSpec((1, _FSZ, D), lambda i, j, m: (m[0, i], j, 0)),
            pl.BlockSpec((1, 1, D), lambda i, j, m: (m[0, i], 0, 0)),
        ],
        out_specs=pl.BlockSpec((TILE, DH), lambda i, j, m: (i, 0)),
        scratch_shapes=[pltpu.VMEM((TILE, D), jnp.float32)],
    )
    return pl.pallas_call(
        _ffn_body,
        grid_spec=grid_spec,
        out_shape=jax.ShapeDtypeStruct((PADDED, DH), jnp.int32),
        compiler_params=pltpu.CompilerParams(
            dimension_semantics=("arbitrary", "arbitrary")),
    )(meta, xs32, W1, b1.reshape(E, 1, DFF), W2, b2.reshape(E, 1, D))


_GATH_W = 128


def _sc_gather(src, idx_row, n_out, bound):
    @pl.kernel(
        out_type=jax.ShapeDtypeStruct((n_out, src.shape[1]), src.dtype),
        mesh=_sc_mesh(),
        scratch_types=[pltpu.VMEM((1, _GATH_W), jnp.int32)])
    def k(x_hbm, i_hbm, o_hbm, idx_scr):
        def body(i_vmem, o_vmem):
            @pl.loop(0, _GATH_W, step=16)
            def _(c):
                sl = (slice(0, 1), pl.ds(c, 16))
                v = i_vmem.at[sl][...]
                idx_scr.at[sl][...] = jnp.minimum(jnp.maximum(v, 0), bound)

            pltpu.sync_copy(x_hbm.at[idx_scr.at[0]], o_vmem)

        pltpu.emit_pipeline(
            body,
            grid=(n_out // _GATH_W,),
            in_specs=[pl.BlockSpec((1, _GATH_W), lambda i: (0, i))],
            out_specs=[pl.BlockSpec((_GATH_W, src.shape[1]),
                                    lambda i: (i, 0))],
            core_axis_name=("c", "s"),
            dimension_semantics=(pltpu.PARALLEL,),
        )(i_hbm, o_hbm)

    return k(src, idx_row)


_ADD_T = 256


def _add_body(g_ref, a_ref, b_ref, o_ref):
    ya = _unpack_bf16(a_ref[...]).astype(jnp.float32)
    yb = _unpack_bf16(b_ref[...]).astype(jnp.float32)
    o_ref[...] = g_ref[:, :1] * ya + g_ref[:, 1:2] * yb


def _pair_add(g2, yg32):
    nt = N // _ADD_T
    return pl.pallas_call(
        _add_body,
        grid=(nt,),
        out_shape=jax.ShapeDtypeStruct((N, D), jnp.float32),
        in_specs=[pl.BlockSpec((_ADD_T, K), lambda i: (i, 0)),
                  pl.BlockSpec((_ADD_T, DH), lambda i: (i, 0)),
                  pl.BlockSpec((_ADD_T, DH), lambda i: (i + nt, 0))],
        out_specs=pl.BlockSpec((_ADD_T, D), lambda i: (i, 0)),
    )(g2, yg32, yg32)


@jax.jit
def kernel(x, Wr, W1, b1, W2, b2):
    x_flat = x.reshape(N, D)
    pk, g2, meta, x32 = _route(x_flat, Wr)

    t_row = (jnp.arange(NK, dtype=jnp.int32) // K).reshape(1, NK)
    p_row = pk.reshape(1, NK)
    xs32 = _sc_dispatch(x32, t_row, p_row)
    ys32 = _ffn(meta, xs32, W1, b1, W2, b2)

    p_cat = jnp.concatenate([pk[:, 0], pk[:, 1]]).reshape(1, NK)
    yg32 = _sc_gather(ys32, p_cat, NK, PADDED - 1)
    out = _pair_add(g2, yg32)
    return out.reshape(B, S, D)


# speedup vs baseline: 1.1922x; 1.1922x over previous
import jax
import jax.numpy as jnp
from jax.experimental import pallas as pl
from jax.experimental.pallas import tpu as pltpu
from jax.experimental.pallas import tpu_sc as plsc

B, S, D, DFF, E, K = 1, 2048, 768, 2048, 8, 2
N = B * S
NK = N * K
DH = D // 2
TILE = 128
PADDED = 5120
GRID_TILES = PADDED // TILE


def _sc_mesh():
    return plsc.VectorSubcoreMesh(core_axis_name="c", subcore_axis_name="s")


def _pack_bf16(a):
    ab = a.astype(jnp.bfloat16)
    hi = jax.lax.bitcast_convert_type(ab[:, :DH], jnp.uint16)
    lo = jax.lax.bitcast_convert_type(ab[:, DH:], jnp.uint16)
    word = (hi.astype(jnp.uint32) << 16) | lo.astype(jnp.uint32)
    return jax.lax.bitcast_convert_type(word, jnp.int32)


def _unpack_bf16(w):
    u = jax.lax.bitcast_convert_type(w, jnp.uint32)
    hi = jax.lax.bitcast_convert_type(
        (u >> 16).astype(jnp.uint16), jnp.bfloat16)
    lo = jax.lax.bitcast_convert_type(
        (u & 0xFFFF).astype(jnp.uint16), jnp.bfloat16)
    return jnp.concatenate([hi, lo], axis=1)


def _router_body(x_ref, wr_ref, pk_ref, g_ref, meta_ref, x32_ref):
    xf = x_ref[...]
    x32_ref[...] = _pack_bf16(xf)
    logits = jax.lax.dot_general(
        xf, wr_ref[...], (((1,), (1,)), ((), ())),
        preferred_element_type=jnp.float32,
        precision=jax.lax.Precision.DEFAULT)
    m = jnp.max(logits, axis=1, keepdims=True)
    p = jnp.exp(logits - m)
    probs = p / jnp.sum(p, axis=1, keepdims=True)

    lane = jax.lax.broadcasted_iota(jnp.int32, (N, E), 1)
    m1 = jnp.max(probs, axis=1, keepdims=True)
    i1 = jnp.min(jnp.where(probs == m1, lane, E), axis=1, keepdims=True)
    probs2 = jnp.where(lane == i1, -1.0, probs)
    m2 = jnp.max(probs2, axis=1, keepdims=True)
    i2 = jnp.min(jnp.where(probs2 == m2, lane, E), axis=1, keepdims=True)
    denom = m1 + m2 + 1e-9
    g_ref[...] = jnp.concatenate([m1 / denom, m2 / denom], axis=1)

    oh1 = (lane == i1).astype(jnp.float32)
    oh2 = (lane == i2).astype(jnp.float32)
    a_all = oh1 + oh2

    r_i = jax.lax.broadcasted_iota(jnp.int32, (N, N), 0)
    c_i = jax.lax.broadcasted_iota(jnp.int32, (N, N), 1)
    ltri = (r_i > c_i).astype(jnp.bfloat16)
    cum = jax.lax.dot_general(
        ltri, a_all.astype(jnp.bfloat16), (((1,), (0,)), ((), ())),
        preferred_element_type=jnp.float32)

    counts = jnp.sum(a_all, axis=0, keepdims=True)
    pc = jnp.ceil(counts * (1.0 / TILE)) * TILE
    e_r = jax.lax.broadcasted_iota(jnp.int32, (E, E), 0)
    e_c = jax.lax.broadcasted_iota(jnp.int32, (E, E), 1)
    up = (e_r < e_c).astype(jnp.float32)
    starts = jax.lax.dot_general(
        pc, up, (((1,), (0,)), ((), ())),
        preferred_element_type=jnp.float32,
        precision=jax.lax.Precision.HIGHEST)

    rank1 = jnp.sum(cum * oh1, axis=1, keepdims=True)
    rank2 = jnp.sum(cum * oh2, axis=1, keepdims=True)
    s1 = jnp.sum(starts * oh1, axis=1, keepdims=True)
    s2 = jnp.sum(starts * oh2, axis=1, keepdims=True)
    pk = jnp.concatenate([s1 + rank1, s2 + rank2], axis=1)
    pk_ref[...] = pk.astype(jnp.int32)

    ones_col = jnp.ones((N, 1), jnp.float32)
    counts_col = jax.lax.dot_general(
        a_all, ones_col, (((0,), (0,)), ((), ())),
        preferred_element_type=jnp.float32)
    pc_col = jnp.ceil(counts_col * (1.0 / TILE)) * TILE
    low = (e_r > e_c).astype(jnp.float32)
    starts_col = jax.lax.dot_general(
        low, pc_col, (((1,), (0,)), ((), ())),
        preferred_element_type=jnp.float32,
        precision=jax.lax.Precision.HIGHEST)
    total = jnp.sum(pc_col)

    tl = jax.lax.broadcasted_iota(jnp.int32, (E, 128), 1).astype(
        jnp.float32) * TILE
    s_cl = jnp.minimum(tl, total - TILE)
    te = jnp.sum((s_cl >= starts_col).astype(jnp.float32), axis=0,
                 keepdims=True) - 1.0
    active = (jax.lax.broadcasted_iota(jnp.int32, (1, 128), 1).astype(
        jnp.float32) * TILE < total).astype(jnp.int32)
    meta_ref[...] = jnp.concatenate([te.astype(jnp.int32), active], axis=0)


def _route(x_flat, Wr):
    return pl.pallas_call(
        _router_body,
        out_shape=(jax.ShapeDtypeStruct((N, K), jnp.int32),
                   jax.ShapeDtypeStruct((N, K), jnp.float32),
                   jax.ShapeDtypeStruct((2, 128), jnp.int32),
                   jax.ShapeDtypeStruct((N, DH), jnp.int32)),
        in_specs=[pl.BlockSpec((N, D), lambda: (0, 0)),
                  pl.BlockSpec((E, D), lambda: (0, 0))],
        out_specs=(pl.BlockSpec((N, K), lambda: (0, 0)),
                   pl.BlockSpec((N, K), lambda: (0, 0)),
                   pl.BlockSpec((2, 128), lambda: (0, 0)),
                   pl.BlockSpec((N, DH), lambda: (0, 0))),
    )(x_flat, Wr)


_DISP_W = 128


def _sc_dispatch(x32, t_row, p_row):
    @pl.kernel(
        out_type=jax.ShapeDtypeStruct((PADDED, DH), jnp.int32),
        mesh=_sc_mesh(),
        scratch_types=[pltpu.VMEM((_DISP_W, DH), jnp.int32)])
    def k(x_hbm, t_hbm, p_hbm, xs_hbm, buf):
        def body(t_vmem, p_vmem):
            pltpu.sync_copy(x_hbm.at[t_vmem.at[0]], buf)
            pltpu.sync_copy(buf, xs_hbm.at[p_vmem.at[0]])

        pltpu.emit_pipeline(
            body,
            grid=(NK // _DISP_W,),
            in_specs=[pl.BlockSpec((1, _DISP_W), lambda i: (0, i)),
                      pl.BlockSpec((1, _DISP_W), lambda i: (0, i))],
            out_specs=[],
            core_axis_name=("c", "s"),
            dimension_semantics=(pltpu.PARALLEL,),
        )(t_hbm, p_hbm)

    return k(x32, t_row, p_row)


def _ffn_body(meta_ref, xs_ref, w1_ref, b1_ref, w2_ref, b2_ref, out_ref):
    i = pl.program_id(0)

    @pl.when(meta_ref[1, i] == 1)
    def _():
        xb = _unpack_bf16(xs_ref[...])
        w1 = w1_ref[0].astype(jnp.bfloat16)
        h = jax.lax.dot_general(
            xb, w1, (((1,), (0,)), ((), ())),
            preferred_element_type=jnp.float32)
        h = h + b1_ref[0]
        h = 0.5 * h * (1.0 + jax.lax.erf(h * 0.7071067811865476))
        w2 = w2_ref[0].astype(jnp.bfloat16)
        y = jax.lax.dot_general(
            h.astype(jnp.bfloat16), w2, (((1,), (0,)), ((), ())),
            preferred_element_type=jnp.float32)
        y = y + b2_ref[0]
        out_ref[...] = _pack_bf16(y)


def _ffn(meta, xs32, W1, b1, W2, b2):
    grid_spec = pltpu.PrefetchScalarGridSpec(
        num_scalar_prefetch=1,
        grid=(GRID_TILES,),
        in_specs=[
            pl.BlockSpec((TILE, DH), lambda i, m: (i, 0)),
            pl.BlockSpec((1, D, DFF), lambda i, m: (m[0, i], 0, 0)),
            pl.BlockSpec((1, 1, DFF), lambda i, m: (m[0, i], 0, 0)),
            pl.BlockSpec((1, DFF, D), lambda i, m: (m[0, i], 0, 0)),
            pl.BlockSpec((1, 1, D), lambda i, m: (m[0, i], 0, 0)),
        ],
        out_specs=pl.BlockSpec((TILE, DH), lambda i, m: (i, 0)),
    )
    return pl.pallas_call(
        _ffn_body,
        grid_spec=grid_spec,
        out_shape=jax.ShapeDtypeStruct((PADDED, DH), jnp.int32),
        compiler_params=pltpu.CompilerParams(
            dimension_semantics=("arbitrary",)),
    )(meta, xs32, W1, b1.reshape(E, 1, DFF), W2, b2.reshape(E, 1, D))


_GATH_W = 128


def _sc_gather(src, idx_row, n_out, bound):
    @pl.kernel(
        out_type=jax.ShapeDtypeStruct((n_out, src.shape[1]), src.dtype),
        mesh=_sc_mesh(),
        scratch_types=[pltpu.VMEM((1, _GATH_W), jnp.int32)])
    def k(x_hbm, i_hbm, o_hbm, idx_scr):
        def body(i_vmem, o_vmem):
            @pl.loop(0, _GATH_W, step=16)
            def _(c):
                sl = (slice(0, 1), pl.ds(c, 16))
                v = i_vmem.at[sl][...]
                idx_scr.at[sl][...] = jnp.minimum(jnp.maximum(v, 0), bound)

            pltpu.sync_copy(x_hbm.at[idx_scr.at[0]], o_vmem)

        pltpu.emit_pipeline(
            body,
            grid=(n_out // _GATH_W,),
            in_specs=[pl.BlockSpec((1, _GATH_W), lambda i: (0, i))],
            out_specs=[pl.BlockSpec((_GATH_W, src.shape[1]),
                                    lambda i: (i, 0))],
            core_axis_name=("c", "s"),
            dimension_semantics=(pltpu.PARALLEL,),
        )(i_hbm, o_hbm)

    return k(src, idx_row)


_ADD_T = 256


def _add_body(g_ref, a_ref, b_ref, o_ref):
    ya = _unpack_bf16(a_ref[...]).astype(jnp.float32)
    yb = _unpack_bf16(b_ref[...]).astype(jnp.float32)
    o_ref[...] = g_ref[:, :1] * ya + g_ref[:, 1:2] * yb


def _pair_add(g2, yg32):
    nt = N // _ADD_T
    return pl.pallas_call(
        _add_body,
        grid=(nt,),
        out_shape=jax.ShapeDtypeStruct((N, D), jnp.float32),
        in_specs=[pl.BlockSpec((_ADD_T, K), lambda i: (i, 0)),
                  pl.BlockSpec((_ADD_T, DH), lambda i: (i, 0)),
                  pl.BlockSpec((_ADD_T, DH), lambda i: (i + nt, 0))],
        out_specs=pl.BlockSpec((_ADD_T, D), lambda i: (i, 0)),
    )(g2, yg32, yg32)


@jax.jit
def kernel(x, Wr, W1, b1, W2, b2):
    x_flat = x.reshape(N, D)
    pk, g2, meta, x32 = _route(x_flat, Wr)

    t_row = (jnp.arange(NK, dtype=jnp.int32) // K).reshape(1, NK)
    p_row = pk.reshape(1, NK)
    xs32 = _sc_dispatch(x32, t_row, p_row)
    ys32 = _ffn(meta, xs32, W1, b1, W2, b2)

    p_cat = jnp.concatenate([pk[:, 0], pk[:, 1]]).reshape(1, NK)
    yg32 = _sc_gather(ys32, p_cat, NK, PADDED - 1)
    out = _pair_add(g2, yg32)
    return out.reshape(B, S, D)


# speedup vs baseline: 1.2274x; 1.0295x over previous
import jax
import jax.numpy as jnp
from jax.experimental import pallas as pl
from jax.experimental.pallas import tpu as pltpu
from jax.experimental.pallas import tpu_sc as plsc

B, S, D, DFF, E, K = 1, 2048, 768, 2048, 8, 2
N = B * S
NK = N * K
DH = D // 2
TILE = 256
PADDED = 6144
GRID_TILES = PADDED // TILE


def _sc_mesh():
    return plsc.VectorSubcoreMesh(core_axis_name="c", subcore_axis_name="s")


def _pack_bf16(a):
    ab = a.astype(jnp.bfloat16)
    hi = jax.lax.bitcast_convert_type(ab[:, :DH], jnp.uint16)
    lo = jax.lax.bitcast_convert_type(ab[:, DH:], jnp.uint16)
    word = (hi.astype(jnp.uint32) << 16) | lo.astype(jnp.uint32)
    return jax.lax.bitcast_convert_type(word, jnp.int32)


def _unpack_bf16(w):
    u = jax.lax.bitcast_convert_type(w, jnp.uint32)
    hi = jax.lax.bitcast_convert_type(
        (u >> 16).astype(jnp.uint16), jnp.bfloat16)
    lo = jax.lax.bitcast_convert_type(
        (u & 0xFFFF).astype(jnp.uint16), jnp.bfloat16)
    return jnp.concatenate([hi, lo], axis=1)


def _router_body(x_ref, wr_ref, lt_ref, pk_ref, g_ref, meta_ref, x32_ref):
    xf = x_ref[...]
    x32_ref[...] = _pack_bf16(xf)
    logits = jax.lax.dot_general(
        xf, wr_ref[...], (((1,), (1,)), ((), ())),
        preferred_element_type=jnp.float32,
        precision=jax.lax.Precision.DEFAULT)
    m = jnp.max(logits, axis=1, keepdims=True)
    p = jnp.exp(logits - m)
    probs = p / jnp.sum(p, axis=1, keepdims=True)

    lane = jax.lax.broadcasted_iota(jnp.int32, (N, E), 1)
    m1 = jnp.max(probs, axis=1, keepdims=True)
    i1 = jnp.min(jnp.where(probs == m1, lane, E), axis=1, keepdims=True)
    probs2 = jnp.where(lane == i1, -1.0, probs)
    m2 = jnp.max(probs2, axis=1, keepdims=True)
    i2 = jnp.min(jnp.where(probs2 == m2, lane, E), axis=1, keepdims=True)
    denom = m1 + m2 + 1e-9
    g_ref[...] = jnp.concatenate([m1 / denom, m2 / denom], axis=1)

    oh1 = (lane == i1).astype(jnp.float32)
    oh2 = (lane == i2).astype(jnp.float32)
    a_all = oh1 + oh2

    cum = jax.lax.dot_general(
        lt_ref[...], a_all.astype(jnp.bfloat16), (((1,), (0,)), ((), ())),
        preferred_element_type=jnp.float32)

    counts = jnp.sum(a_all, axis=0, keepdims=True)
    pc = jnp.ceil(counts * (1.0 / TILE)) * TILE
    e_r = jax.lax.broadcasted_iota(jnp.int32, (E, E), 0)
    e_c = jax.lax.broadcasted_iota(jnp.int32, (E, E), 1)
    up = (e_r < e_c).astype(jnp.float32)
    starts = jax.lax.dot_general(
        pc, up, (((1,), (0,)), ((), ())),
        preferred_element_type=jnp.float32,
        precision=jax.lax.Precision.HIGHEST)

    rank1 = jnp.sum(cum * oh1, axis=1, keepdims=True)
    rank2 = jnp.sum(cum * oh2, axis=1, keepdims=True)
    s1 = jnp.sum(starts * oh1, axis=1, keepdims=True)
    s2 = jnp.sum(starts * oh2, axis=1, keepdims=True)
    pk = jnp.concatenate([s1 + rank1, s2 + rank2], axis=1)
    pk_ref[...] = pk.astype(jnp.int32)

    ones_col = jnp.ones((N, 1), jnp.float32)
    counts_col = jax.lax.dot_general(
        a_all, ones_col, (((0,), (0,)), ((), ())),
        preferred_element_type=jnp.float32)
    pc_col = jnp.ceil(counts_col * (1.0 / TILE)) * TILE
    low = (e_r > e_c).astype(jnp.float32)
    starts_col = jax.lax.dot_general(
        low, pc_col, (((1,), (0,)), ((), ())),
        preferred_element_type=jnp.float32,
        precision=jax.lax.Precision.HIGHEST)
    total = jnp.sum(pc_col)

    tl = jax.lax.broadcasted_iota(jnp.int32, (E, 128), 1).astype(
        jnp.float32) * TILE
    s_cl = jnp.minimum(tl, total - TILE)
    te = jnp.sum((s_cl >= starts_col).astype(jnp.float32), axis=0,
                 keepdims=True) - 1.0
    active = (jax.lax.broadcasted_iota(jnp.int32, (1, 128), 1).astype(
        jnp.float32) * TILE < total).astype(jnp.int32)
    meta_ref[...] = jnp.concatenate([te.astype(jnp.int32), active], axis=0)


def _route(x_flat, Wr):
    ltri = jnp.tril(jnp.ones((N, N), jnp.bfloat16), -1)
    return pl.pallas_call(
        _router_body,
        out_shape=(jax.ShapeDtypeStruct((N, K), jnp.int32),
                   jax.ShapeDtypeStruct((N, K), jnp.float32),
                   jax.ShapeDtypeStruct((2, 128), jnp.int32),
                   jax.ShapeDtypeStruct((N, DH), jnp.int32)),
        in_specs=[pl.BlockSpec((N, D), lambda: (0, 0)),
                  pl.BlockSpec((E, D), lambda: (0, 0)),
                  pl.BlockSpec((N, N), lambda: (0, 0))],
        out_specs=(pl.BlockSpec((N, K), lambda: (0, 0)),
                   pl.BlockSpec((N, K), lambda: (0, 0)),
                   pl.BlockSpec((2, 128), lambda: (0, 0)),
                   pl.BlockSpec((N, DH), lambda: (0, 0))),
    )(x_flat, Wr, ltri)


_DISP_W = 128


def _sc_dispatch(x32, t_row, p_row):
    @pl.kernel(
        out_type=jax.ShapeDtypeStruct((PADDED, DH), jnp.int32),
        mesh=_sc_mesh(),
        scratch_types=[pltpu.VMEM((_DISP_W, DH), jnp.int32)])
    def k(x_hbm, t_hbm, p_hbm, xs_hbm, buf):
        def body(t_vmem, p_vmem):
            pltpu.sync_copy(x_hbm.at[t_vmem.at[0]], buf)
            pltpu.sync_copy(buf, xs_hbm.at[p_vmem.at[0]])

        pltpu.emit_pipeline(
            body,
            grid=(NK // _DISP_W,),
            in_specs=[pl.BlockSpec((1, _DISP_W), lambda i: (0, i)),
                      pl.BlockSpec((1, _DISP_W), lambda i: (0, i))],
            out_specs=[],
            core_axis_name=("c", "s"),
            dimension_semantics=(pltpu.PARALLEL,),
        )(t_hbm, p_hbm)

    return k(x32, t_row, p_row)


def _ffn_body(meta_ref, xs_ref, w1_ref, b1_ref, w2_ref, b2_ref, out_ref):
    i = pl.program_id(0)

    @pl.when(meta_ref[1, i] == 1)
    def _():
        xb = _unpack_bf16(xs_ref[...])
        w1 = w1_ref[0].astype(jnp.bfloat16)
        h = jax.lax.dot_general(
            xb, w1, (((1,), (0,)), ((), ())),
            preferred_element_type=jnp.float32)
        h = h + b1_ref[0]
        h = 0.5 * h * (1.0 + jax.lax.erf(h * 0.7071067811865476))
        w2 = w2_ref[0].astype(jnp.bfloat16)
        y = jax.lax.dot_general(
            h.astype(jnp.bfloat16), w2, (((1,), (0,)), ((), ())),
            preferred_element_type=jnp.float32)
        y = y + b2_ref[0]
        out_ref[...] = _pack_bf16(y)


def _ffn(meta, xs32, W1, b1, W2, b2):
    grid_spec = pltpu.PrefetchScalarGridSpec(
        num_scalar_prefetch=1,
        grid=(GRID_TILES,),
        in_specs=[
            pl.BlockSpec((TILE, DH), lambda i, m: (i, 0)),
            pl.BlockSpec((1, D, DFF), lambda i, m: (m[0, i], 0, 0)),
            pl.BlockSpec((1, 1, DFF), lambda i, m: (m[0, i], 0, 0)),
            pl.BlockSpec((1, DFF, D), lambda i, m: (m[0, i], 0, 0)),
            pl.BlockSpec((1, 1, D), lambda i, m: (m[0, i], 0, 0)),
        ],
        out_specs=pl.BlockSpec((TILE, DH), lambda i, m: (i, 0)),
    )
    return pl.pallas_call(
        _ffn_body,
        grid_spec=grid_spec,
        out_shape=jax.ShapeDtypeStruct((PADDED, DH), jnp.int32),
        compiler_params=pltpu.CompilerParams(
            dimension_semantics=("arbitrary",)),
    )(meta, xs32, W1, b1.reshape(E, 1, DFF), W2, b2.reshape(E, 1, D))


_GATH_W = 128


def _sc_gather(src, idx_row, n_out, bound):
    @pl.kernel(
        out_type=jax.ShapeDtypeStruct((n_out, src.shape[1]), src.dtype),
        mesh=_sc_mesh(),
        scratch_types=[pltpu.VMEM((1, _GATH_W), jnp.int32)])
    def k(x_hbm, i_hbm, o_hbm, idx_scr):
        def body(i_vmem, o_vmem):
            @pl.loop(0, _GATH_W, step=16)
            def _(c):
                sl = (slice(0, 1), pl.ds(c, 16))
                v = i_vmem.at[sl][...]
                idx_scr.at[sl][...] = jnp.minimum(jnp.maximum(v, 0), bound)

            pltpu.sync_copy(x_hbm.at[idx_scr.at[0]], o_vmem)

        pltpu.emit_pipeline(
            body,
            grid=(n_out // _GATH_W,),
            in_specs=[pl.BlockSpec((1, _GATH_W), lambda i: (0, i))],
            out_specs=[pl.BlockSpec((_GATH_W, src.shape[1]),
                                    lambda i: (i, 0))],
            core_axis_name=("c", "s"),
            dimension_semantics=(pltpu.PARALLEL,),
        )(i_hbm, o_hbm)

    return k(src, idx_row)


_ADD_T = 256


def _add_body(g_ref, a_ref, b_ref, o_ref):
    ya = _unpack_bf16(a_ref[...]).astype(jnp.float32)
    yb = _unpack_bf16(b_ref[...]).astype(jnp.float32)
    o_ref[...] = g_ref[:, :1] * ya + g_ref[:, 1:2] * yb


def _pair_add(g2, yg32):
    nt = N // _ADD_T
    return pl.pallas_call(
        _add_body,
        grid=(nt,),
        out_shape=jax.ShapeDtypeStruct((N, D), jnp.float32),
        in_specs=[pl.BlockSpec((_ADD_T, K), lambda i: (i, 0)),
                  pl.BlockSpec((_ADD_T, DH), lambda i: (i, 0)),
                  pl.BlockSpec((_ADD_T, DH), lambda i: (i + nt, 0))],
        out_specs=pl.BlockSpec((_ADD_T, D), lambda i: (i, 0)),
    )(g2, yg32, yg32)


@jax.jit
def kernel(x, Wr, W1, b1, W2, b2):
    x_flat = x.reshape(N, D)
    pk, g2, meta, x32 = _route(x_flat, Wr)

    t_row = (jnp.arange(NK, dtype=jnp.int32) // K).reshape(1, NK)
    p_row = pk.reshape(1, NK)
    xs32 = _sc_dispatch(x32, t_row, p_row)
    ys32 = _ffn(meta, xs32, W1, b1, W2, b2)

    p_cat = jnp.concatenate([pk[:, 0], pk[:, 1]]).reshape(1, NK)
    yg32 = _sc_gather(ys32, p_cat, NK, PADDED - 1)
    out = _pair_add(g2, yg32)
    return out.reshape(B, S, D)


# speedup vs baseline: 1.3186x; 1.0743x over previous
import jax
import jax.numpy as jnp
from jax.experimental import pallas as pl
from jax.experimental.pallas import tpu as pltpu
from jax.experimental.pallas import tpu_sc as plsc

B, S, D, DFF, E, K = 1, 2048, 768, 2048, 8, 2
N = B * S
NK = N * K
DH = D // 2
TILE = 256
PADDED = 6144
GRID_TILES = PADDED // TILE


def _sc_mesh():
    return plsc.VectorSubcoreMesh(core_axis_name="c", subcore_axis_name="s")


def _pack_bf16(a):
    ab = a.astype(jnp.bfloat16)
    hi = jax.lax.bitcast_convert_type(ab[:, :DH], jnp.uint16)
    lo = jax.lax.bitcast_convert_type(ab[:, DH:], jnp.uint16)
    word = (hi.astype(jnp.uint32) << 16) | lo.astype(jnp.uint32)
    return jax.lax.bitcast_convert_type(word, jnp.int32)


def _unpack_bf16(w):
    u = jax.lax.bitcast_convert_type(w, jnp.uint32)
    hi = jax.lax.bitcast_convert_type(
        (u >> 16).astype(jnp.uint16), jnp.bfloat16)
    lo = jax.lax.bitcast_convert_type(
        (u & 0xFFFF).astype(jnp.uint16), jnp.bfloat16)
    return jnp.concatenate([hi, lo], axis=1)


def _router_body(x_ref, wr_ref, pk_ref, g_ref, meta_ref, x32_ref):
    xf = x_ref[...]
    x32_ref[...] = _pack_bf16(xf)
    logits = jax.lax.dot_general(
        xf, wr_ref[...], (((1,), (1,)), ((), ())),
        preferred_element_type=jnp.float32,
        precision=jax.lax.Precision.DEFAULT)
    m = jnp.max(logits, axis=1, keepdims=True)
    p = jnp.exp(logits - m)
    probs = p / jnp.sum(p, axis=1, keepdims=True)

    lane = jax.lax.broadcasted_iota(jnp.int32, (N, E), 1)
    m1 = jnp.max(probs, axis=1, keepdims=True)
    i1 = jnp.min(jnp.where(probs == m1, lane, E), axis=1, keepdims=True)
    probs2 = jnp.where(lane == i1, -1.0, probs)
    m2 = jnp.max(probs2, axis=1, keepdims=True)
    i2 = jnp.min(jnp.where(probs2 == m2, lane, E), axis=1, keepdims=True)
    denom = m1 + m2 + 1e-9
    g_ref[...] = jnp.concatenate([m1 / denom, m2 / denom], axis=1)

    oh1 = (lane == i1).astype(jnp.float32)
    oh2 = (lane == i2).astype(jnp.float32)
    a_all = oh1 + oh2

    nch = N // 128
    a3 = a_all.reshape(nch, 128, E)
    r_i = jax.lax.broadcasted_iota(jnp.int32, (128, 128), 0)
    c_i = jax.lax.broadcasted_iota(jnp.int32, (128, 128), 1)
    lt128 = jnp.broadcast_to(
        ((r_i > c_i).astype(jnp.bfloat16))[None], (nch, 128, 128))
    intra = jax.lax.dot_general(
        lt128, a3.astype(jnp.bfloat16),
        (((2,), (1,)), ((0,), (0,))),
        preferred_element_type=jnp.float32)
    ct = jnp.sum(a3, axis=1)
    n_r = jax.lax.broadcasted_iota(jnp.int32, (nch, nch), 0)
    n_c = jax.lax.broadcasted_iota(jnp.int32, (nch, nch), 1)
    ltn = (n_r > n_c).astype(jnp.float32)
    off = jax.lax.dot_general(
        ltn, ct, (((1,), (0,)), ((), ())),
        preferred_element_type=jnp.float32,
        precision=jax.lax.Precision.HIGHEST)
    cum = (intra + off[:, None, :]).reshape(N, E)

    counts = jnp.sum(a_all, axis=0, keepdims=True)
    pc = jnp.ceil(counts * (1.0 / TILE)) * TILE
    e_r = jax.lax.broadcasted_iota(jnp.int32, (E, E), 0)
    e_c = jax.lax.broadcasted_iota(jnp.int32, (E, E), 1)
    up = (e_r < e_c).astype(jnp.float32)
    starts = jax.lax.dot_general(
        pc, up, (((1,), (0,)), ((), ())),
        preferred_element_type=jnp.float32,
        precision=jax.lax.Precision.HIGHEST)

    rank1 = jnp.sum(cum * oh1, axis=1, keepdims=True)
    rank2 = jnp.sum(cum * oh2, axis=1, keepdims=True)
    s1 = jnp.sum(starts * oh1, axis=1, keepdims=True)
    s2 = jnp.sum(starts * oh2, axis=1, keepdims=True)
    pk = jnp.concatenate([s1 + rank1, s2 + rank2], axis=1)
    pk_ref[...] = pk.astype(jnp.int32)

    ones_col = jnp.ones((N, 1), jnp.float32)
    counts_col = jax.lax.dot_general(
        a_all, ones_col, (((0,), (0,)), ((), ())),
        preferred_element_type=jnp.float32)
    pc_col = jnp.ceil(counts_col * (1.0 / TILE)) * TILE
    low = (e_r > e_c).astype(jnp.float32)
    starts_col = jax.lax.dot_general(
        low, pc_col, (((1,), (0,)), ((), ())),
        preferred_element_type=jnp.float32,
        precision=jax.lax.Precision.HIGHEST)
    total = jnp.sum(pc_col)

    tl = jax.lax.broadcasted_iota(jnp.int32, (E, 128), 1).astype(
        jnp.float32) * TILE
    s_cl = jnp.minimum(tl, total - TILE)
    te = jnp.sum((s_cl >= starts_col).astype(jnp.float32), axis=0,
                 keepdims=True) - 1.0
    active = (jax.lax.broadcasted_iota(jnp.int32, (1, 128), 1).astype(
        jnp.float32) * TILE < total).astype(jnp.int32)
    meta_ref[...] = jnp.concatenate([te.astype(jnp.int32), active], axis=0)


def _route(x_flat, Wr):
    return pl.pallas_call(
        _router_body,
        out_shape=(jax.ShapeDtypeStruct((N, K), jnp.int32),
                   jax.ShapeDtypeStruct((N, K), jnp.float32),
                   jax.ShapeDtypeStruct((2, 128), jnp.int32),
                   jax.ShapeDtypeStruct((N, DH), jnp.int32)),
        in_specs=[pl.BlockSpec((N, D), lambda: (0, 0)),
                  pl.BlockSpec((E, D), lambda: (0, 0))],
        out_specs=(pl.BlockSpec((N, K), lambda: (0, 0)),
                   pl.BlockSpec((N, K), lambda: (0, 0)),
                   pl.BlockSpec((2, 128), lambda: (0, 0)),
                   pl.BlockSpec((N, DH), lambda: (0, 0))),
    )(x_flat, Wr)


_DISP_W = 128


def _sc_dispatch(x32, t_row, p_row):
    @pl.kernel(
        out_type=jax.ShapeDtypeStruct((PADDED, DH), jnp.int32),
        mesh=_sc_mesh(),
        scratch_types=[pltpu.VMEM((_DISP_W, DH), jnp.int32)])
    def k(x_hbm, t_hbm, p_hbm, xs_hbm, buf):
        def body(t_vmem, p_vmem):
            pltpu.sync_copy(x_hbm.at[t_vmem.at[0]], buf)
            pltpu.sync_copy(buf, xs_hbm.at[p_vmem.at[0]])

        pltpu.emit_pipeline(
            body,
            grid=(NK // _DISP_W,),
            in_specs=[pl.BlockSpec((1, _DISP_W), lambda i: (0, i)),
                      pl.BlockSpec((1, _DISP_W), lambda i: (0, i))],
            out_specs=[],
            core_axis_name=("c", "s"),
            dimension_semantics=(pltpu.PARALLEL,),
        )(t_hbm, p_hbm)

    return k(x32, t_row, p_row)


def _ffn_body(meta_ref, xs_ref, w1_ref, b1_ref, w2_ref, b2_ref, out_ref):
    i = pl.program_id(0)

    @pl.when(meta_ref[1, i] == 1)
    def _():
        xb = _unpack_bf16(xs_ref[...])
        w1 = w1_ref[0].astype(jnp.bfloat16)
        h = jax.lax.dot_general(
            xb, w1, (((1,), (0,)), ((), ())),
            preferred_element_type=jnp.float32)
        h = h + b1_ref[0]
        h = 0.5 * h * (1.0 + jax.lax.erf(h * 0.7071067811865476))
        w2 = w2_ref[0].astype(jnp.bfloat16)
        y = jax.lax.dot_general(
            h.astype(jnp.bfloat16), w2, (((1,), (0,)), ((), ())),
            preferred_element_type=jnp.float32)
        y = y + b2_ref[0]
        out_ref[...] = _pack_bf16(y)


def _ffn(meta, xs32, W1, b1, W2, b2):
    grid_spec = pltpu.PrefetchScalarGridSpec(
        num_scalar_prefetch=1,
        grid=(GRID_TILES,),
        in_specs=[
            pl.BlockSpec((TILE, DH), lambda i, m: (i, 0)),
            pl.BlockSpec((1, D, DFF), lambda i, m: (m[0, i], 0, 0)),
            pl.BlockSpec((1, 1, DFF), lambda i, m: (m[0, i], 0, 0)),
            pl.BlockSpec((1, DFF, D), lambda i, m: (m[0, i], 0, 0)),
            pl.BlockSpec((1, 1, D), lambda i, m: (m[0, i], 0, 0)),
        ],
        out_specs=pl.BlockSpec((TILE, DH), lambda i, m: (i, 0)),
    )
    return pl.pallas_call(
        _ffn_body,
        grid_spec=grid_spec,
        out_shape=jax.ShapeDtypeStruct((PADDED, DH), jnp.int32),
        compiler_params=pltpu.CompilerParams(
            dimension_semantics=("arbitrary",)),
    )(meta, xs32, W1, b1.reshape(E, 1, DFF), W2, b2.reshape(E, 1, D))


_GATH_W = 128


def _sc_gather(src, idx_row, n_out, bound):
    @pl.kernel(
        out_type=jax.ShapeDtypeStruct((n_out, src.shape[1]), src.dtype),
        mesh=_sc_mesh(),
        scratch_types=[pltpu.VMEM((1, _GATH_W), jnp.int32)])
    def k(x_hbm, i_hbm, o_hbm, idx_scr):
        def body(i_vmem, o_vmem):
            @pl.loop(0, _GATH_W, step=16)
            def _(c):
                sl = (slice(0, 1), pl.ds(c, 16))
                v = i_vmem.at[sl][...]
                idx_scr.at[sl][...] = jnp.minimum(jnp.maximum(v, 0), bound)

            pltpu.sync_copy(x_hbm.at[idx_scr.at[0]], o_vmem)

        pltpu.emit_pipeline(
            body,
            grid=(n_out // _GATH_W,),
            in_specs=[pl.BlockSpec((1, _GATH_W), lambda i: (0, i))],
            out_specs=[pl.BlockSpec((_GATH_W, src.shape[1]),
                                    lambda i: (i, 0))],
            core_axis_name=("c", "s"),
            dimension_semantics=(pltpu.PARALLEL,),
        )(i_hbm, o_hbm)

    return k(src, idx_row)


_ADD_T = 256


def _add_body(g_ref, a_ref, b_ref, o_ref):
    ya = _unpack_bf16(a_ref[...]).astype(jnp.float32)
    yb = _unpack_bf16(b_ref[...]).astype(jnp.float32)
    o_ref[...] = g_ref[:, :1] * ya + g_ref[:, 1:2] * yb


def _pair_add(g2, yg32):
    nt = N // _ADD_T
    return pl.pallas_call(
        _add_body,
        grid=(nt,),
        out_shape=jax.ShapeDtypeStruct((N, D), jnp.float32),
        in_specs=[pl.BlockSpec((_ADD_T, K), lambda i: (i, 0)),
                  pl.BlockSpec((_ADD_T, DH), lambda i: (i, 0)),
                  pl.BlockSpec((_ADD_T, DH), lambda i: (i + nt, 0))],
        out_specs=pl.BlockSpec((_ADD_T, D), lambda i: (i, 0)),
    )(g2, yg32, yg32)


@jax.jit
def kernel(x, Wr, W1, b1, W2, b2):
    x_flat = x.reshape(N, D)
    pk, g2, meta, x32 = _route(x_flat, Wr)

    t_row = (jnp.arange(NK, dtype=jnp.int32) // K).reshape(1, NK)
    p_row = pk.reshape(1, NK)
    xs32 = _sc_dispatch(x32, t_row, p_row)
    ys32 = _ffn(meta, xs32, W1, b1, W2, b2)

    p_cat = jnp.concatenate([pk[:, 0], pk[:, 1]]).reshape(1, NK)
    yg32 = _sc_gather(ys32, p_cat, NK, PADDED - 1)
    out = _pair_add(g2, yg32)
    return out.reshape(B, S, D)
